# baseline (speedup 1.0000x reference)
import jax
import jax.numpy as jnp
from jax.experimental import pallas as pl
from jax.experimental.pallas import tpu as pltpu

_B, _C, _H, _W = 4, 64, 48, 48
_N = _H * _W
_K = _B * _C
_R = 2 * _K
_THR = 0.5
_EPS = 1e-8
_TH = 8
_NS = _TH * _W
_TI = 192
_DB = _TI * _H
_NB = _N // _TI


def _round_scale_t(x):
    xr = x.astype(jnp.bfloat16).astype(jnp.float32)
    parts = []
    for g in range(_K // _C):
        blk = x[g * _C:(g + 1) * _C, :]
        ss = jnp.sum(blk * blk, axis=0, keepdims=True)
        inv = 1.0 / jnp.maximum(jnp.sqrt(ss), _EPS)
        parts.append(xr[g * _C:(g + 1) * _C, :] * inv)
    return jnp.transpose(jnp.concatenate(parts, axis=0))


def _fused_kernel(d_ref, y_ref, z_ref, zp_ref, yp_ref,
                  acc_ref, cnt_ref, vh_s, vl_s):
    i = pl.program_id(0)

    @pl.when(i == 0)
    def _():
        for t in range(_H // _TH):
            vt_zp = _round_scale_t(
                zp_ref[:, t * _TH:(t + 1) * _TH, :].reshape(_K, _NS))
            vt_yp = _round_scale_t(
                yp_ref[:, t * _TH:(t + 1) * _TH, :].reshape(_K, _NS))
            vh_zp = vt_zp.astype(jnp.bfloat16)
            vh_yp = vt_yp.astype(jnp.bfloat16)
            sl = pl.ds(t * _NS, _NS)
            vh_s[sl, :_K] = vh_zp
            vh_s[sl, _K:] = vh_yp
            vl_s[sl, :_K] = (vt_zp - vh_zp.astype(jnp.float32)).astype(jnp.bfloat16)
            vl_s[sl, _K:] = (vt_yp - vh_yp.astype(jnp.float32)).astype(jnp.bfloat16)

    hb = _TI // _W
    ut = jnp.concatenate(
        [_round_scale_t(y_ref[:, pl.ds(i * hb, hb), :].reshape(_K, _TI)),
         _round_scale_t(z_ref[:, pl.ds(i * hb, hb), :].reshape(_K, _TI))],
        axis=1)
    mask = d_ref[...].reshape(_TI, _H, _W) < _THR
    mb = mask.astype(jnp.bfloat16).reshape(_TI, _N)
    w = (jnp.dot(mb, vh_s[...], preferred_element_type=jnp.float32)
         + jnp.dot(mb, vl_s[...], preferred_element_type=jnp.float32))
    acc_ref[0, 0, 0] = jnp.sum(w * ut)
    cnt_ref[0, 0, 0] = jnp.sum(mask.astype(jnp.float32))


@jax.jit
def kernel(y, yp, z, zp, distances):
    y3 = y.reshape(_K, _H, _W)
    z3 = z.reshape(_K, _H, _W)
    zp3 = zp.reshape(_K, _H, _W)
    yp3 = yp.reshape(_K, _H, _W)
    df = distances.reshape(_N * _H, _W)

    acc, cnt = pl.pallas_call(
        _fused_kernel,
        grid=(_NB,),
        in_specs=[
            pl.BlockSpec((_DB, _W), lambda i: (i, 0)),
            pl.BlockSpec((_K, _H, _W), lambda i: (0, 0, 0)),
            pl.BlockSpec((_K, _H, _W), lambda i: (0, 0, 0)),
            pl.BlockSpec((_K, _H, _W), lambda i: (0, 0, 0)),
            pl.BlockSpec((_K, _H, _W), lambda i: (0, 0, 0)),
        ],
        out_specs=[
            pl.BlockSpec((1, 1, 1), lambda i: (i, 0, 0), memory_space=pltpu.SMEM),
            pl.BlockSpec((1, 1, 1), lambda i: (i, 0, 0), memory_space=pltpu.SMEM),
        ],
        out_shape=[
            jax.ShapeDtypeStruct((_NB, 1, 1), jnp.float32),
            jax.ShapeDtypeStruct((_NB, 1, 1), jnp.float32),
        ],
        scratch_shapes=[
            pltpu.VMEM((_N, _R), jnp.bfloat16),
            pltpu.VMEM((_N, _R), jnp.bfloat16),
        ],
        compiler_params=pltpu.CompilerParams(
            dimension_semantics=("arbitrary",)),
    )(df, y3, z3, zp3, yp3)

    return -jnp.sum(acc) / (jnp.sum(cnt) * jnp.float32(_B))


# speedup vs baseline: 1.0031x; 1.0031x over previous
import jax
import jax.numpy as jnp
from jax.experimental import pallas as pl
from jax.experimental.pallas import tpu as pltpu

_B, _C, _H, _W = 4, 64, 48, 48
_N = _H * _W
_K = _B * _C
_R = 2 * _K
_THR = 0.5
_EPS = 1e-8
_TH = 8
_NS = _TH * _W
_NJ = _H // _TH


def _round_scale_t(x):
    xr = x.astype(jnp.bfloat16).astype(jnp.float32)
    parts = []
    for g in range(_K // _C):
        blk = x[g * _C:(g + 1) * _C, :]
        ss = jnp.sum(blk * blk, axis=0, keepdims=True)
        inv = 1.0 / jnp.maximum(jnp.sqrt(ss), _EPS)
        parts.append(xr[g * _C:(g + 1) * _C, :] * inv)
    return jnp.transpose(jnp.concatenate(parts, axis=0))


def _fused_kernel(d_ref, y_ref, z_ref, zp_ref, yp_ref,
                  acc_ref, cnt_ref, w_s, ut_s):
    j = pl.program_id(0)

    vt_zp = _round_scale_t(zp_ref[...].reshape(_K, _NS))
    vt_yp = _round_scale_t(yp_ref[...].reshape(_K, _NS))
    vh = jnp.concatenate([vt_zp, vt_yp], axis=1).astype(jnp.bfloat16)
    vl = (jnp.concatenate([vt_zp, vt_yp], axis=1)
          - vh.astype(jnp.float32)).astype(jnp.bfloat16)

    mask = d_ref[...] < _THR
    mb = mask.astype(jnp.bfloat16).reshape(_N, _NS)
    wj = (jnp.dot(mb, vh, preferred_element_type=jnp.float32)
          + jnp.dot(mb, vl, preferred_element_type=jnp.float32))
    cj = jnp.sum(mask.astype(jnp.float32))

    ut_s[pl.ds(j * _NS, _NS), :] = jnp.concatenate(
        [_round_scale_t(y_ref[...].reshape(_K, _NS)),
         _round_scale_t(z_ref[...].reshape(_K, _NS))], axis=1)

    @pl.when(j == 0)
    def _():
        w_s[...] = wj
        cnt_ref[0, 0] = cj

    @pl.when(j > 0)
    def _():
        w_s[...] += wj
        cnt_ref[0, 0] += cj

    @pl.when(j == _NJ - 1)
    def _():
        acc_ref[0, 0] = jnp.sum(w_s[...] * ut_s[...])


@jax.jit
def kernel(y, yp, z, zp, distances):
    y3 = y.reshape(_K, _H, _W)
    z3 = z.reshape(_K, _H, _W)
    zp3 = zp.reshape(_K, _H, _W)
    yp3 = yp.reshape(_K, _H, _W)
    d3 = distances.reshape(_N, _H, _W)

    acc, cnt = pl.pallas_call(
        _fused_kernel,
        grid=(_NJ,),
        in_specs=[
            pl.BlockSpec((_N, _TH, _W), lambda j: (0, j, 0)),
            pl.BlockSpec((_K, _TH, _W), lambda j: (0, j, 0)),
            pl.BlockSpec((_K, _TH, _W), lambda j: (0, j, 0)),
            pl.BlockSpec((_K, _TH, _W), lambda j: (0, j, 0)),
            pl.BlockSpec((_K, _TH, _W), lambda j: (0, j, 0)),
        ],
        out_specs=[
            pl.BlockSpec(memory_space=pltpu.SMEM),
            pl.BlockSpec(memory_space=pltpu.SMEM),
        ],
        out_shape=[
            jax.ShapeDtypeStruct((1, 1), jnp.float32),
            jax.ShapeDtypeStruct((1, 1), jnp.float32),
        ],
        scratch_shapes=[
            pltpu.VMEM((_N, _R), jnp.float32),
            pltpu.VMEM((_N, _R), jnp.float32),
        ],
        compiler_params=pltpu.CompilerParams(
            dimension_semantics=("arbitrary",)),
    )(d3, y3, z3, zp3, yp3)

    return -acc[0, 0] / (cnt[0, 0] * jnp.float32(_B))


# speedup vs baseline: 1.0265x; 1.0233x over previous
import jax
import jax.numpy as jnp
from jax.experimental import pallas as pl
from jax.experimental.pallas import tpu as pltpu

_B, _C, _H, _W = 4, 64, 48, 48
_N = _H * _W
_K = _B * _C
_R = 2 * _K
_THR = 0.5
_EPS = 1e-8
_TH = 16
_NS = _TH * _W
_TI = 384
_HB = _TI // _W
_NB = _N // _TI


def _round_scale_t(x, t):
    xr = x.astype(jnp.bfloat16).astype(jnp.float32)
    parts = []
    for g in range(_K // _C):
        blk = x[g * _C:(g + 1) * _C, :]
        ss = jnp.sum(blk * blk, axis=0, keepdims=True)
        inv = 1.0 / jnp.maximum(jnp.sqrt(ss), _EPS)
        parts.append(xr[g * _C:(g + 1) * _C, :] * inv)
    return jnp.transpose(jnp.concatenate(parts, axis=0))


def _prep_kernel(zp_ref, yp_ref, vh_ref, vl_ref):
    vt_zp = _round_scale_t(zp_ref[...].reshape(_K, _NS), _NS)
    vt_yp = _round_scale_t(yp_ref[...].reshape(_K, _NS), _NS)
    vh_zp = vt_zp.astype(jnp.bfloat16)
    vh_yp = vt_yp.astype(jnp.bfloat16)
    vh_ref[:, :_K] = vh_zp
    vh_ref[:, _K:] = vh_yp
    vl_ref[:, :_K] = (vt_zp - vh_zp.astype(jnp.float32)).astype(jnp.bfloat16)
    vl_ref[:, _K:] = (vt_yp - vh_yp.astype(jnp.float32)).astype(jnp.bfloat16)


def _main_kernel(d_ref, y_ref, z_ref, vh_ref, vl_ref, acc_ref, cnt_ref):
    ut = jnp.concatenate(
        [_round_scale_t(y_ref[...].reshape(_K, _TI), _TI),
         _round_scale_t(z_ref[...].reshape(_K, _TI), _TI)], axis=1)
    mask = d_ref[...] < _THR
    mb = mask.astype(jnp.bfloat16).reshape(_TI, _N)
    w = (jnp.dot(mb, vh_ref[...], preferred_element_type=jnp.float32)
         + jnp.dot(mb, vl_ref[...], preferred_element_type=jnp.float32))
    acc_ref[0, 0, 0] = jnp.sum(w * ut)
    cnt_ref[0, 0, 0] = jnp.sum(mask.astype(jnp.float32))


@jax.jit
def kernel(y, yp, z, zp, distances):
    y3 = y.reshape(_K, _H, _W)
    z3 = z.reshape(_K, _H, _W)
    zp3 = zp.reshape(_K, _H, _W)
    yp3 = yp.reshape(_K, _H, _W)
    d3 = distances.reshape(_N, _H, _W)

    vh, vl = pl.pallas_call(
        _prep_kernel,
        grid=(_H // _TH,),
        in_specs=[
            pl.BlockSpec((_K, _TH, _W), lambda t: (0, t, 0)),
            pl.BlockSpec((_K, _TH, _W), lambda t: (0, t, 0)),
        ],
        out_specs=[
            pl.BlockSpec((_NS, _R), lambda t: (t, 0)),
            pl.BlockSpec((_NS, _R), lambda t: (t, 0)),
        ],
        out_shape=[
            jax.ShapeDtypeStruct((_N, _R), jnp.bfloat16),
            jax.ShapeDtypeStruct((_N, _R), jnp.bfloat16),
        ],
        compiler_params=pltpu.CompilerParams(
            dimension_semantics=("parallel",)),
    )(zp3, yp3)

    acc, cnt = pl.pallas_call(
        _main_kernel,
        grid=(_NB,),
        in_specs=[
            pl.BlockSpec((_TI, _H, _W), lambda i: (i, 0, 0)),
            pl.BlockSpec((_K, _HB, _W), lambda i: (0, i, 0)),
            pl.BlockSpec((_K, _HB, _W), lambda i: (0, i, 0)),
            pl.BlockSpec((_N, _R), lambda i: (0, 0)),
            pl.BlockSpec((_N, _R), lambda i: (0, 0)),
        ],
        out_specs=[
            pl.BlockSpec((1, 1, 1), lambda i: (i, 0, 0), memory_space=pltpu.SMEM),
            pl.BlockSpec((1, 1, 1), lambda i: (i, 0, 0), memory_space=pltpu.SMEM),
        ],
        out_shape=[
            jax.ShapeDtypeStruct((_NB, 1, 1), jnp.float32),
            jax.ShapeDtypeStruct((_NB, 1, 1), jnp.float32),
        ],
        compiler_params=pltpu.CompilerParams(
            dimension_semantics=("parallel",)),
    )(d3, y3, z3, vh, vl)

    return -jnp.sum(acc) / (jnp.sum(cnt) * jnp.float32(_B))


# speedup vs baseline: 1.0981x; 1.0697x over previous
import jax
import jax.numpy as jnp
from jax.experimental import pallas as pl
from jax.experimental.pallas import tpu as pltpu

_B, _C, _H, _W = 4, 64, 48, 48
_N = _H * _W
_K = _B * _C
_R = 2 * _K
_THR = 0.5
_EPS = 1e-8
_TH = 16
_NS = _TH * _W
_NP = _H // _TH
_TI = 384
_HB = _TI // _W
_NB = _N // _TI


def _round_scale_t(x):
    xr = x.astype(jnp.bfloat16).astype(jnp.float32)
    parts = []
    for g in range(_K // _C):
        blk = x[g * _C:(g + 1) * _C, :]
        ss = jnp.sum(blk * blk, axis=0, keepdims=True)
        inv = 1.0 / jnp.maximum(jnp.sqrt(ss), _EPS)
        parts.append(xr[g * _C:(g + 1) * _C, :] * inv)
    return jnp.transpose(jnp.concatenate(parts, axis=0))


def _fused_kernel(d_ref, y_ref, z_ref, zp_ref, yp_ref,
                  acc_ref, cnt_ref, vh_s, vl_s):
    i = pl.program_id(0)

    @pl.when(i < _NP)
    def _():
        vt_zp = _round_scale_t(zp_ref[...].reshape(_K, _NS))
        vt_yp = _round_scale_t(yp_ref[...].reshape(_K, _NS))
        vh_zp = vt_zp.astype(jnp.bfloat16)
        vh_yp = vt_yp.astype(jnp.bfloat16)
        sl = pl.ds(i * _NS, _NS)
        vh_s[sl, :_K] = vh_zp
        vh_s[sl, _K:] = vh_yp
        vl_s[sl, :_K] = (vt_zp - vh_zp.astype(jnp.float32)).astype(jnp.bfloat16)
        vl_s[sl, _K:] = (vt_yp - vh_yp.astype(jnp.float32)).astype(jnp.bfloat16)

    @pl.when(i >= _NP)
    def _():
        ut = jnp.concatenate(
            [_round_scale_t(y_ref[...].reshape(_K, _TI)),
             _round_scale_t(z_ref[...].reshape(_K, _TI))], axis=1)
        mask = d_ref[...] < _THR
        mb = mask.astype(jnp.bfloat16).reshape(_TI, _N)
        w = (jnp.dot(mb, vh_s[...], preferred_element_type=jnp.float32)
             + jnp.dot(mb, vl_s[...], preferred_element_type=jnp.float32))
        acc_ref[0, 0, 0] = jnp.sum(w * ut)
        cnt_ref[0, 0, 0] = jnp.sum(mask.astype(jnp.float32))


@jax.jit
def kernel(y, yp, z, zp, distances):
    y3 = y.reshape(_K, _H, _W)
    z3 = z.reshape(_K, _H, _W)
    zp3 = zp.reshape(_K, _H, _W)
    yp3 = yp.reshape(_K, _H, _W)
    d3 = distances.reshape(_N, _H, _W)

    band = lambda i: jnp.maximum(i - _NP, 0)
    chunk = lambda i: jnp.minimum(i, _NP - 1)

    acc, cnt = pl.pallas_call(
        _fused_kernel,
        grid=(_NP + _NB,),
        in_specs=[
            pl.BlockSpec((_TI, _H, _W), lambda i: (band(i), 0, 0)),
            pl.BlockSpec((_K, _HB, _W), lambda i: (0, band(i), 0)),
            pl.BlockSpec((_K, _HB, _W), lambda i: (0, band(i), 0)),
            pl.BlockSpec((_K, _TH, _W), lambda i: (0, chunk(i), 0)),
            pl.BlockSpec((_K, _TH, _W), lambda i: (0, chunk(i), 0)),
        ],
        out_specs=[
            pl.BlockSpec((1, 1, 1), lambda i: (band(i), 0, 0),
                         memory_space=pltpu.SMEM),
            pl.BlockSpec((1, 1, 1), lambda i: (band(i), 0, 0),
                         memory_space=pltpu.SMEM),
        ],
        out_shape=[
            jax.ShapeDtypeStruct((_NB, 1, 1), jnp.float32),
            jax.ShapeDtypeStruct((_NB, 1, 1), jnp.float32),
        ],
        scratch_shapes=[
            pltpu.VMEM((_N, _R), jnp.bfloat16),
            pltpu.VMEM((_N, _R), jnp.bfloat16),
        ],
        compiler_params=pltpu.CompilerParams(
            dimension_semantics=("arbitrary",)),
    )(d3, y3, z3, zp3, yp3)

    return -jnp.sum(acc) / (jnp.sum(cnt) * jnp.float32(_B))
